# baseline (speedup 1.0000x reference)
import functools

import jax
import jax.numpy as jnp
from jax import lax
from jax.experimental import pallas as pl
from jax.experimental.pallas import tpu as pltpu
from jax.experimental.pallas import tpu_sc as plsc

N = 10000
D = 128
E = 320000
NC, NS = 2, 16
NW = NC * NS
C = 128
CHUNKS = 80
EPT = C * CHUNKS
E_PAD = NW * EPT
N_PAD = 10240
ROWS_PER_TILE = N_PAD // NS
MW = 3 * C
RB = 2
MB = 4


_SPLAT_DNUMS = lax.GatherDimensionNumbers(
    offset_dims=(), collapsed_slice_dims=(0,), start_index_map=(0,))


def _sc_scatter_kernel(src_x_hbm, meta_hbm, zeros_hbm, out_hbm, *scr):
    meta_v = scr[0:MB]
    sidx_v = scr[MB:MB + RB]
    didx_v = scr[MB + RB:MB + 2 * RB]
    rows_v = scr[MB + 2 * RB:MB + 3 * RB]
    acc_sh = scr[MB + 3 * RB]
    sem_m = scr[MB + 3 * RB + 1:2 * MB + 3 * RB + 1]
    sem_g = scr[2 * MB + 3 * RB + 1:2 * MB + 4 * RB + 1]
    sem_s = scr[2 * MB + 4 * RB + 1:2 * MB + 5 * RB + 1]

    c = lax.axis_index("c")
    s = lax.axis_index("s")
    base = (c * NS + s) * CHUNKS

    with jax.named_scope("acc_zero"):
        pltpu.sync_copy(zeros_hbm, acc_sh.at[pl.ds(s * ROWS_PER_TILE, ROWS_PER_TILE)])
        plsc.subcore_barrier()

    def issue_meta(g, q):
        pltpu.async_copy(meta_hbm.at[pl.ds((base + g) * MW, MW)], meta_v[q],
                         sem_m[q])

    def wait_meta(q):
        pltpu.make_async_copy(meta_hbm.at[pl.ds(0, MW)], meta_v[q], sem_m[q]).wait()

    def wait_scatter(b):
        pltpu.make_async_copy(rows_v[b], acc_sh.at[didx_v[b]], sem_s[b]).wait()

    def extract_and_gather(g, b, q):
        for k in range(C // 16):
            sl = pl.ds(k * 16, 16)
            sidx_v[b][sl] = lax.convert_element_type(meta_v[q][sl], jnp.int32)
            didx_v[b][sl] = lax.convert_element_type(
                meta_v[q][pl.ds(C + k * 16, 16)], jnp.int32)
        pltpu.async_copy(src_x_hbm.at[sidx_v[b]], rows_v[b], sem_g[b])

    def step(g, j, wait_sc=True, do_next=True, do_meta=True):
        b, q = j % RB, j % MB
        b1, q1, q3 = (j + 1) % RB, (j + 1) % MB, (j + 3) % MB
        if do_next:
            wait_meta(q1)
            if wait_sc:
                wait_scatter(b1)
            extract_and_gather(g + 1, b1, q1)
        if do_meta:
            issue_meta(g + 3, q3)
        pltpu.make_async_copy(src_x_hbm.at[sidx_v[b]], rows_v[b], sem_g[b]).wait()

        rows = rows_v[b]
        meta = meta_v[q]

        def row_block(t, _):
            w16 = meta[pl.ds(2 * C + t * 16, 16)]
            for u in range(16):
                wspl = lax.gather(
                    w16, jnp.full((16, 1), u, jnp.int32), _SPLAT_DNUMS, (1,),
                    mode=lax.GatherScatterMode.PROMISE_IN_BOUNDS)
                for k in range(D // 16):
                    sl = pl.ds(k * 16, 16)
                    rows[t * 16 + u, sl] = rows[t * 16 + u, sl] * wspl
            return 0

        lax.fori_loop(0, C // 16, row_block, 0)
        pltpu.async_copy(rows, acc_sh.at[didx_v[b]], sem_s[b], add=True)

    issue_meta(0, 0)
    issue_meta(1, 1)
    issue_meta(2, 2)
    wait_meta(0)
    extract_and_gather(0, 0, 0)

    with jax.named_scope("edge_pipeline"):
        for j in range(MB):
            step(j, j, wait_sc=(j >= 1))

        def outer_body(k, _):
            for j in range(MB):
                step(k * MB + j, j)
            return 0

        lax.fori_loop(1, CHUNKS // MB - 1, outer_body, 0)

        for j in range(MB):
            step(CHUNKS - MB + j, j, do_next=(j < 3), do_meta=(j < 1))

        wait_scatter(0)
        wait_scatter(1)

    with jax.named_scope("writeout"):
        plsc.subcore_barrier()

        pltpu.sync_copy(
            acc_sh.at[pl.ds(s * ROWS_PER_TILE, ROWS_PER_TILE)],
            out_hbm.at[c, pl.ds(s * ROWS_PER_TILE, ROWS_PER_TILE)])


_sc_scratch = (
    [pltpu.VMEM((MW,), jnp.float32) for _ in range(MB)]
    + [pltpu.VMEM((C,), jnp.int32) for _ in range(RB)]
    + [pltpu.VMEM((C,), jnp.int32) for _ in range(RB)]
    + [pltpu.VMEM((C, D), jnp.float32) for _ in range(RB)]
    + [pltpu.VMEM_SHARED((N_PAD, D), jnp.float32)]
    + [pltpu.SemaphoreType.DMA for _ in range(MB + 2 * RB)]
)

_sc_call = functools.partial(
    pl.kernel,
    out_type=jax.ShapeDtypeStruct((NC, N_PAD, D), jnp.float32),
    mesh=plsc.VectorSubcoreMesh(core_axis_name="c", subcore_axis_name="s"),
    scratch_types=_sc_scratch,
)


def _sc_scatter(src_x, meta, zeros):
    return _sc_call(_sc_scatter_kernel)(src_x, meta, zeros)


def _combine_body(p_ref, o_ref):
    o_ref[...] = p_ref[0] + p_ref[1]


BT = 32
BE = BT * C
T_CHUNKS = E_PAD // C
LAST_REAL_BLOCK = E // BE


def _meta_body(ei_ref, w_ref, o_ref):
    i = pl.program_id(0)
    s = ei_ref[0, :].astype(jnp.float32).reshape(BT, C)
    d = ei_ref[1, :].astype(jnp.float32).reshape(BT, C)
    w = w_ref[:, 0].reshape(BT, C)
    real = jnp.concatenate([s, d, w], axis=1)
    col = jax.lax.broadcasted_iota(jnp.int32, (BT, MW), 1)
    chunk = jax.lax.broadcasted_iota(jnp.int32, (BT, MW), 0) + i * BT
    e = chunk * C + col % C
    spread = (((e - E) * 131) % N).astype(jnp.float32)
    padval = jnp.where(col < 2 * C, spread, 0.0)
    o_ref[...] = jnp.where(e < E, real, padval)


def _build_meta(edge_index, edge_weight):
    return pl.pallas_call(
        _meta_body,
        out_shape=jax.ShapeDtypeStruct((T_CHUNKS, MW), jnp.float32),
        grid=(T_CHUNKS // BT,),
        in_specs=[
            pl.BlockSpec((2, BE), lambda i: (0, jnp.minimum(i, LAST_REAL_BLOCK))),
            pl.BlockSpec((BE, 1), lambda i: (jnp.minimum(i, LAST_REAL_BLOCK), 0)),
        ],
        out_specs=pl.BlockSpec((BT, MW), lambda i: (i, 0)),
    )(edge_index, edge_weight).reshape(-1)


def kernel(src_x, dst_x, edge_index, edge_weight):
    meta = _build_meta(edge_index.astype(jnp.int32), edge_weight)
    zeros = jnp.zeros((ROWS_PER_TILE, D), jnp.float32)

    partials = _sc_scatter(src_x, meta, zeros)

    BR = 1000
    return pl.pallas_call(
        _combine_body,
        out_shape=jax.ShapeDtypeStruct((N, D), jnp.float32),
        grid=(N // BR,),
        in_specs=[pl.BlockSpec((NC, BR, D), lambda i: (0, i, 0))],
        out_specs=pl.BlockSpec((BR, D), lambda i: (i, 0)),
    )(partials)


# speedup vs baseline: 1.7369x; 1.7369x over previous
import functools

import jax
import jax.numpy as jnp
from jax import lax
from jax.experimental import pallas as pl
from jax.experimental.pallas import tpu as pltpu
from jax.experimental.pallas import tpu_sc as plsc

N = 10000
D = 128
E = 320000
NC, NS = 2, 16
NW = NC * NS
C = 128
CHUNKS = 80
EPT = C * CHUNKS
E_PAD = NW * EPT
N_PAD = 10240
ROWS_PER_TILE = N_PAD // NS
MW = 3 * C
RB = 2
MB = 4


_SPLAT_DNUMS = lax.GatherDimensionNumbers(
    offset_dims=(), collapsed_slice_dims=(0,), start_index_map=(0,))


def _sc_scatter_kernel(src_x_hbm, meta_hbm, zeros_hbm, out_hbm, *scr):
    meta_v = scr[0:MB]
    sidx_v = scr[MB:MB + RB]
    didx_v = scr[MB + RB:MB + 2 * RB]
    rows_v = scr[MB + 2 * RB:MB + 3 * RB]
    acc_sh = scr[MB + 3 * RB]
    sem_m = scr[MB + 3 * RB + 1:2 * MB + 3 * RB + 1]
    sem_g = scr[2 * MB + 3 * RB + 1:2 * MB + 4 * RB + 1]
    sem_s = scr[2 * MB + 4 * RB + 1:2 * MB + 5 * RB + 1]

    c = lax.axis_index("c")
    s = lax.axis_index("s")
    base = (c * NS + s) * CHUNKS

    with jax.named_scope("acc_zero"):
        pltpu.sync_copy(zeros_hbm, acc_sh.at[pl.ds(s * ROWS_PER_TILE, ROWS_PER_TILE)])
        plsc.subcore_barrier()

    def issue_meta(g, q):
        pltpu.async_copy(meta_hbm.at[pl.ds((base + g) * MW, MW)], meta_v[q],
                         sem_m[q])

    def wait_meta(q):
        pltpu.make_async_copy(meta_hbm.at[pl.ds(0, MW)], meta_v[q], sem_m[q]).wait()

    def wait_scatter(b):
        pltpu.make_async_copy(rows_v[b], acc_sh.at[didx_v[b]], sem_s[b]).wait()

    def extract_and_gather(g, b, q):
        for k in range(C // 16):
            sl = pl.ds(k * 16, 16)
            sidx_v[b][sl] = lax.convert_element_type(meta_v[q][sl], jnp.int32)
            didx_v[b][sl] = lax.convert_element_type(
                meta_v[q][pl.ds(C + k * 16, 16)], jnp.int32)
        pltpu.async_copy(src_x_hbm.at[sidx_v[b]], rows_v[b], sem_g[b])

    def step(g, j, wait_sc=True, do_next=True, do_meta=True):
        b, q = j % RB, j % MB
        b1, q1, q3 = (j + 1) % RB, (j + 1) % MB, (j + 3) % MB
        if do_next:
            wait_meta(q1)
            if wait_sc:
                wait_scatter(b1)
            extract_and_gather(g + 1, b1, q1)
        if do_meta:
            issue_meta(g + 3, q3)
        pltpu.make_async_copy(src_x_hbm.at[sidx_v[b]], rows_v[b], sem_g[b]).wait()

        rows = rows_v[b]
        meta = meta_v[q]

        def row_block(t, _):
            w16 = meta[pl.ds(2 * C + t * 16, 16)]
            for u in range(16):
                wspl = lax.gather(
                    w16, jnp.full((16, 1), u, jnp.int32), _SPLAT_DNUMS, (1,),
                    mode=lax.GatherScatterMode.PROMISE_IN_BOUNDS)
                for k in range(D // 16):
                    sl = pl.ds(k * 16, 16)
                    rows[t * 16 + u, sl] = rows[t * 16 + u, sl] * wspl
            return 0

        lax.fori_loop(0, C // 16, row_block, 0)
        pltpu.async_copy(rows, acc_sh.at[didx_v[b]], sem_s[b], add=True)

    issue_meta(0, 0)
    issue_meta(1, 1)
    issue_meta(2, 2)
    wait_meta(0)
    extract_and_gather(0, 0, 0)

    with jax.named_scope("edge_pipeline"):
        for j in range(MB):
            step(j, j, wait_sc=(j >= 1))

        def outer_body(k, _):
            for j in range(MB):
                step(k * MB + j, j)
            return 0

        lax.fori_loop(1, CHUNKS // MB - 1, outer_body, 0)

        for j in range(MB):
            step(CHUNKS - MB + j, j, do_next=(j < 3), do_meta=(j < 1))

        wait_scatter(0)
        wait_scatter(1)

    with jax.named_scope("writeout"):
        plsc.subcore_barrier()

        pltpu.sync_copy(
            acc_sh.at[pl.ds(s * ROWS_PER_TILE, ROWS_PER_TILE)],
            out_hbm.at[c, pl.ds(s * ROWS_PER_TILE, ROWS_PER_TILE)])


_sc_scratch = (
    [pltpu.VMEM((MW,), jnp.float32) for _ in range(MB)]
    + [pltpu.VMEM((C,), jnp.int32) for _ in range(RB)]
    + [pltpu.VMEM((C,), jnp.int32) for _ in range(RB)]
    + [pltpu.VMEM((C, D), jnp.float32) for _ in range(RB)]
    + [pltpu.VMEM_SHARED((N_PAD, D), jnp.float32)]
    + [pltpu.SemaphoreType.DMA for _ in range(MB + 2 * RB)]
)

_sc_call = functools.partial(
    pl.kernel,
    out_type=jax.ShapeDtypeStruct((NC, N_PAD, D), jnp.float32),
    mesh=plsc.VectorSubcoreMesh(core_axis_name="c", subcore_axis_name="s"),
    scratch_types=_sc_scratch,
)


def _sc_scatter(src_x, meta, zeros):
    return _sc_call(_sc_scatter_kernel)(src_x, meta, zeros)


def _combine_body(p_ref, o_ref):
    o_ref[...] = p_ref[0] + p_ref[1]


def kernel(src_x, dst_x, edge_index, edge_weight):
    pad = E_PAD - E
    spread = (jnp.arange(pad, dtype=jnp.int32) * 131) % N
    sidx = jnp.concatenate([edge_index[0].astype(jnp.int32), spread])
    didx = jnp.concatenate([edge_index[1].astype(jnp.int32), spread])
    w = jnp.concatenate([edge_weight[:, 0], jnp.zeros((pad,), jnp.float32)])
    T = E_PAD // C
    meta = jnp.concatenate([
        sidx.astype(jnp.float32).reshape(T, C),
        didx.astype(jnp.float32).reshape(T, C),
        w.reshape(T, C),
    ], axis=1).reshape(-1)
    zeros = jnp.zeros((ROWS_PER_TILE, D), jnp.float32)

    partials = _sc_scatter(src_x, meta, zeros)

    BR = 1000
    return pl.pallas_call(
        _combine_body,
        out_shape=jax.ShapeDtypeStruct((N, D), jnp.float32),
        grid=(N // BR,),
        in_specs=[pl.BlockSpec((NC, BR, D), lambda i: (0, i, 0))],
        out_specs=pl.BlockSpec((BR, D), lambda i: (i, 0)),
    )(partials)


# speedup vs baseline: 1.8077x; 1.0408x over previous
import functools

import jax
import jax.numpy as jnp
from jax import lax
from jax.experimental import pallas as pl
from jax.experimental.pallas import tpu as pltpu
from jax.experimental.pallas import tpu_sc as plsc

N = 10000
D = 128
E = 320000
NC, NS = 2, 16
NW = NC * NS
C = 128
CHUNKS = 80
EPT = C * CHUNKS
E_PAD = NW * EPT
N_PAD = 10240
ROWS_PER_TILE = N_PAD // NS
RB = 2
IR = 8

_SPLAT_DNUMS = lax.GatherDimensionNumbers(
    offset_dims=(), collapsed_slice_dims=(0,), start_index_map=(0,))


def _sc_scatter_kernel(src_x_hbm, eidx_hbm, w_hbm, zeros_hbm, out_hbm, *scr):
    sidx_v = scr[0:IR]
    didx_v = scr[IR:2 * IR]
    w_v = scr[2 * IR:3 * IR]
    rows_v = scr[3 * IR:3 * IR + RB]
    acc_sh = scr[3 * IR + RB]
    sem_idx = scr[3 * IR + RB + 1:4 * IR + RB + 1]
    sem_g = scr[4 * IR + RB + 1:4 * IR + 2 * RB + 1]
    sem_s = scr[4 * IR + 2 * RB + 1:4 * IR + 3 * RB + 1]

    c = lax.axis_index("c")
    s = lax.axis_index("s")
    base = (c * NS + s) * EPT

    def issue_idx(g, q):
        off = base + g * C
        pltpu.async_copy(eidx_hbm.at[0, pl.ds(off, C)], sidx_v[q], sem_idx[q])
        pltpu.async_copy(eidx_hbm.at[1, pl.ds(off, C)], didx_v[q], sem_idx[q])
        pltpu.async_copy(w_hbm.at[pl.ds(off, C)], w_v[q], sem_idx[q])

    def wait_idx(q):
        pltpu.make_async_copy(eidx_hbm.at[0, pl.ds(0, C)], sidx_v[q], sem_idx[q]).wait()
        pltpu.make_async_copy(eidx_hbm.at[1, pl.ds(0, C)], didx_v[q], sem_idx[q]).wait()
        pltpu.make_async_copy(w_hbm.at[pl.ds(0, C)], w_v[q], sem_idx[q]).wait()

    def wait_scatter(b, q):
        pltpu.make_async_copy(rows_v[b], acc_sh.at[didx_v[q]], sem_s[b]).wait()

    def step(g, j, wait_sc=True, do_next=True, do_idx=True):
        b, q = j % RB, j % IR
        b1, q1 = (j + 1) % RB, (j + 1) % IR
        q4 = (j + 4) % IR
        if do_next:
            wait_idx(q1)
            if wait_sc:
                wait_scatter(b1, q1)
            pltpu.async_copy(src_x_hbm.at[sidx_v[q1]], rows_v[b1], sem_g[b1])
        pltpu.make_async_copy(src_x_hbm.at[sidx_v[q]], rows_v[b], sem_g[b]).wait()
        if do_idx:
            issue_idx(g + 4, q4)

        rows = rows_v[b]
        wv = w_v[q]

        def row_block(t, _):
            w16 = wv[pl.ds(t * 16, 16)]
            for u in range(16):
                wspl = lax.gather(
                    w16, jnp.full((16, 1), u, jnp.int32), _SPLAT_DNUMS, (1,),
                    mode=lax.GatherScatterMode.PROMISE_IN_BOUNDS)
                for k in range(D // 16):
                    sl = pl.ds(k * 16, 16)
                    rows[t * 16 + u, sl] = rows[t * 16 + u, sl] * wspl
            return 0

        lax.fori_loop(0, C // 16, row_block, 0)
        pltpu.async_copy(rows, acc_sh.at[didx_v[q]], sem_s[b], add=True)

    for t in range(4):
        issue_idx(t, t)
    wait_idx(0)
    pltpu.async_copy(src_x_hbm.at[sidx_v[0]], rows_v[0], sem_g[0])

    with jax.named_scope("acc_zero"):
        pltpu.sync_copy(zeros_hbm,
                        acc_sh.at[pl.ds(s * ROWS_PER_TILE, ROWS_PER_TILE)])
        plsc.subcore_barrier()

    with jax.named_scope("edge_pipeline"):
        for j in range(IR):
            step(j, j, wait_sc=(j >= 1))

        def outer_body(k, _):
            for j in range(IR):
                step(k * IR + j, j)
            return 0

        lax.fori_loop(1, CHUNKS // IR - 1, outer_body, 0)

        for j in range(IR):
            step(CHUNKS - IR + j, j, do_next=(j < 7), do_idx=(j < 4))

        wait_scatter(0, 6)
        wait_scatter(1, 7)

    with jax.named_scope("writeout"):
        plsc.subcore_barrier()
        pltpu.sync_copy(
            acc_sh.at[pl.ds(s * ROWS_PER_TILE, ROWS_PER_TILE)],
            out_hbm.at[c, pl.ds(s * ROWS_PER_TILE, ROWS_PER_TILE)])


_sc_scratch = (
    [pltpu.VMEM((C,), jnp.int32) for _ in range(IR)]
    + [pltpu.VMEM((C,), jnp.int32) for _ in range(IR)]
    + [pltpu.VMEM((C,), jnp.float32) for _ in range(IR)]
    + [pltpu.VMEM((C, D), jnp.float32) for _ in range(RB)]
    + [pltpu.VMEM_SHARED((N_PAD, D), jnp.float32)]
    + [pltpu.SemaphoreType.DMA for _ in range(IR + 2 * RB)]
)

_sc_call = functools.partial(
    pl.kernel,
    out_type=jax.ShapeDtypeStruct((NC, N_PAD, D), jnp.float32),
    mesh=plsc.VectorSubcoreMesh(core_axis_name="c", subcore_axis_name="s"),
    scratch_types=_sc_scratch,
)


def _sc_scatter(src_x, eidx, w, zeros):
    return _sc_call(_sc_scatter_kernel)(src_x, eidx, w, zeros)


def _combine_body(p_ref, o_ref):
    o_ref[...] = p_ref[0] + p_ref[1]


def kernel(src_x, dst_x, edge_index, edge_weight):
    pad = E_PAD - E
    spread = (jnp.arange(pad, dtype=jnp.int32) * 131) % N
    eidx = jnp.concatenate(
        [edge_index.astype(jnp.int32), jnp.stack([spread, spread])], axis=1)
    w = jnp.concatenate([edge_weight[:, 0], jnp.zeros((pad,), jnp.float32)])
    zeros = jnp.zeros((ROWS_PER_TILE, D), jnp.float32)

    partials = _sc_scatter(src_x, eidx, w, zeros)

    BR = 1000
    return pl.pallas_call(
        _combine_body,
        out_shape=jax.ShapeDtypeStruct((N, D), jnp.float32),
        grid=(N // BR,),
        in_specs=[pl.BlockSpec((NC, BR, D), lambda i: (0, i, 0))],
        out_specs=pl.BlockSpec((BR, D), lambda i: (i, 0)),
    )(partials)
